# Initial kernel scaffold; baseline (speedup 1.0000x reference)
import jax
import jax.numpy as jnp
from jax.experimental import pallas as pl


def kernel(feats, table, modality_id):
    raise NotImplementedError("write your pallas kernel here")


# speedup vs baseline: 4.0670x; 4.0670x over previous
import jax
import jax.numpy as jnp
from jax.experimental import pallas as pl
from jax.experimental.pallas import tpu as pltpu

_PAD_ROWS = 8


def _add_kernel(idx_ref, feats_ref, table_ref, out_ref):
    i = idx_ref[0]
    tbl = table_ref[...]
    rows = jax.lax.broadcasted_iota(jnp.int32, (_PAD_ROWS, 1), 0)
    mask = (rows == i).astype(tbl.dtype)
    row = jnp.sum(tbl * mask, axis=0, keepdims=True)
    out_ref[...] = feats_ref[...] + row


def kernel(feats, table, modality_id):
    B, S, D = feats.shape
    N = B * S
    x = feats.reshape(N, D)
    n_rows = table.shape[0]
    tbl = jnp.pad(table, ((0, _PAD_ROWS - n_rows), (0, 0)))
    idx = jnp.asarray(modality_id, jnp.int32).reshape(1)

    rows_per_block = 2048
    grid = (N // rows_per_block,)

    out = pl.pallas_call(
        _add_kernel,
        grid_spec=pltpu.PrefetchScalarGridSpec(
            num_scalar_prefetch=1,
            grid=grid,
            in_specs=[
                pl.BlockSpec((rows_per_block, D), lambda i, idx_ref: (i, 0)),
                pl.BlockSpec((_PAD_ROWS, D), lambda i, idx_ref: (0, 0)),
            ],
            out_specs=pl.BlockSpec((rows_per_block, D), lambda i, idx_ref: (i, 0)),
        ),
        out_shape=jax.ShapeDtypeStruct((N, D), feats.dtype),
        compiler_params=pltpu.CompilerParams(
            dimension_semantics=("arbitrary",),
        ),
    )(idx, x, tbl)
    return out.reshape(B, S, D)
